# Initial kernel scaffold; baseline (speedup 1.0000x reference)
import jax
import jax.numpy as jnp
from jax.experimental import pallas as pl


def kernel(x, edge_index, batch, W1, b1, W2, b2, W3, b3, W4, b4):
    raise NotImplementedError("write your pallas kernel here")


# speedup vs baseline: 11.4967x; 11.4967x over previous
import functools

import jax
import jax.numpy as jnp
from jax import lax
from jax.experimental import pallas as pl
from jax.experimental.pallas import tpu as pltpu
from jax.experimental.pallas import tpu_sc as plsc

N = 10000
NP = 10240
E = 320000
D = 128

NC = 2
NS = 16
CHUNK = 128
EPC = E // NC
NCHUNKS = EPC // CHUNK
ROWS_PER_TILE = NP // NS
DEGW = 128

assert EPC % CHUNK == 0 and NP % NS == 0 and ROWS_PER_TILE % 8 == 0

_MESH = dict(core_axis_name="c", subcore_axis_name="s",
             num_cores=NC, num_subcores=NS)


@functools.partial(
    pl.kernel,
    out_type=jax.ShapeDtypeStruct((NC, NP, D), jnp.float32),
    mesh=plsc.VectorSubcoreMesh(**_MESH),
    scratch_types=[
        pltpu.VMEM_SHARED((NP, D), jnp.float32),
        pltpu.VMEM((CHUNK,), jnp.int32),
        pltpu.VMEM((CHUNK,), jnp.int32),
        pltpu.VMEM((CHUNK, D), jnp.float32),
        pltpu.SemaphoreType.DMA,
    ],
)
def _prop(y_hbm, src_hbm, dst_hbm, zeros_hbm, out_hbm,
          acc, srcv, dstv, rows, sem):
    c = lax.axis_index("c")
    s = lax.axis_index("s")
    r0 = s * ROWS_PER_TILE
    pltpu.sync_copy(zeros_hbm, acc.at[pl.ds(r0, ROWS_PER_TILE)])
    plsc.subcore_barrier()
    nt = (NCHUNKS - s + NS - 1) // NS
    ebase = c * EPC

    def body(i, carry):
        off = ebase + (s + i * NS) * CHUNK
        pltpu.sync_copy(src_hbm.at[pl.ds(off, CHUNK)], srcv)
        pltpu.sync_copy(dst_hbm.at[pl.ds(off, CHUNK)], dstv)
        pltpu.async_copy(y_hbm.at[srcv], rows, sem).wait()
        pltpu.sync_copy(rows, acc.at[dstv], add=True)
        return carry

    lax.fori_loop(0, nt, body, 0)
    plsc.subcore_barrier()
    pltpu.sync_copy(acc.at[pl.ds(r0, ROWS_PER_TILE)],
                    out_hbm.at[c, pl.ds(r0, ROWS_PER_TILE)])


@functools.partial(
    pl.kernel,
    out_type=jax.ShapeDtypeStruct((NC, NP, DEGW), jnp.float32),
    mesh=plsc.VectorSubcoreMesh(**_MESH),
    scratch_types=[
        pltpu.VMEM_SHARED((NP, DEGW), jnp.float32),
        pltpu.VMEM((CHUNK,), jnp.int32),
        pltpu.VMEM((CHUNK, DEGW), jnp.float32),
    ],
)
def _deg(dst_hbm, ones_hbm, zeros_hbm, out_hbm, acc, dstv, onesv):
    c = lax.axis_index("c")
    s = lax.axis_index("s")
    r0 = s * ROWS_PER_TILE
    pltpu.sync_copy(ones_hbm, onesv)
    pltpu.sync_copy(zeros_hbm, acc.at[pl.ds(r0, ROWS_PER_TILE)])
    plsc.subcore_barrier()
    nt = (NCHUNKS - s + NS - 1) // NS
    ebase = c * EPC

    def body(i, carry):
        off = ebase + (s + i * NS) * CHUNK
        pltpu.sync_copy(dst_hbm.at[pl.ds(off, CHUNK)], dstv)
        pltpu.sync_copy(onesv, acc.at[dstv], add=True)
        return carry

    lax.fori_loop(0, nt, body, 0)
    plsc.subcore_barrier()
    pltpu.sync_copy(acc.at[pl.ds(r0, ROWS_PER_TILE)],
                    out_hbm.at[c, pl.ds(r0, ROWS_PER_TILE)])


BLK = 2048
_GRID = NP // BLK


def _dinv(d0, d1):
    return lax.rsqrt(d0[:, :1] + d1[:, :1] + 1.0)


def _tc1_body(x_ref, w_ref, d0_ref, d1_ref, y_ref):
    dinv = _dinv(d0_ref[...], d1_ref[...])
    y_ref[...] = dinv * jnp.dot(x_ref[...], w_ref[...],
                                preferred_element_type=jnp.float32)


def _tcmid_body(p0_ref, p1_ref, yp_ref, d0_ref, d1_ref, b_ref, w_ref, y_ref):
    dinv = _dinv(d0_ref[...], d1_ref[...])
    h = jnp.maximum(dinv * (p0_ref[...] + p1_ref[...] + yp_ref[...])
                    + b_ref[...], 0.0)
    y_ref[...] = dinv * jnp.dot(h, w_ref[...],
                                preferred_element_type=jnp.float32)


def _tcpre4_body(p0_ref, p1_ref, yp_ref, d0_ref, d1_ref, b_ref, y_ref):
    dinv = _dinv(d0_ref[...], d1_ref[...])
    y_ref[...] = dinv * jnp.maximum(
        dinv * (p0_ref[...] + p1_ref[...] + yp_ref[...]) + b_ref[...], 0.0)


def _tcfin_body(p0_ref, p1_ref, yp_ref, d0_ref, d1_ref, b_ref, w_ref, o_ref):
    dinv = _dinv(d0_ref[...], d1_ref[...])
    g = dinv * (p0_ref[...] + p1_ref[...] + yp_ref[...])
    o = jnp.dot(g, w_ref[...], preferred_element_type=jnp.float32) + b_ref[...]
    m = jnp.maximum(o[:, :1], o[:, 1:2])
    lse = m + jnp.log(jnp.exp(o[:, :1] - m) + jnp.exp(o[:, 1:2] - m))
    o_ref[...] = o - lse


def _row_spec(w):
    return pl.BlockSpec((BLK, w), lambda i: (i, 0))


def _full_spec(r, w):
    return pl.BlockSpec((r, w), lambda i: (0, 0))


def _tc_call(body, in_specs, out_w=D):
    return pl.pallas_call(
        body,
        grid=(_GRID,),
        in_specs=in_specs,
        out_specs=_row_spec(out_w),
        out_shape=jax.ShapeDtypeStruct((NP, out_w), jnp.float32),
    )


def kernel(x, edge_index, batch, W1, b1, W2, b2, W3, b3, W4, b4):
    src = edge_index[0]
    dst = edge_index[1]
    xp = jnp.pad(x, ((0, NP - N), (0, 0)))
    zeros_p = jnp.zeros((ROWS_PER_TILE, D), jnp.float32)
    zeros_d = jnp.zeros((ROWS_PER_TILE, DEGW), jnp.float32)
    ones_d = jnp.ones((CHUNK, DEGW), jnp.float32)
    W4p = jnp.zeros((D, D), jnp.float32).at[:, :2].set(W4)
    b4p = jnp.zeros((1, D), jnp.float32).at[0, :2].set(b4)

    degp = _deg(dst, ones_d, zeros_d)
    d0, d1 = degp[0], degp[1]

    tc1 = _tc_call(_tc1_body,
                   [_row_spec(D), _full_spec(D, D),
                    _row_spec(DEGW), _row_spec(DEGW)])
    tcmid = _tc_call(_tcmid_body,
                     [_row_spec(D)] * 3 + [_row_spec(DEGW)] * 2
                     + [_full_spec(1, D), _full_spec(D, D)])
    tcpre4 = _tc_call(_tcpre4_body,
                      [_row_spec(D)] * 3 + [_row_spec(DEGW)] * 2
                      + [_full_spec(1, D)])
    tcfin = _tc_call(_tcfin_body,
                     [_row_spec(D)] * 3 + [_row_spec(DEGW)] * 2
                     + [_full_spec(1, D), _full_spec(D, D)])

    y1 = tc1(xp, W1, d0, d1)
    p = _prop(y1, src, dst, zeros_p)
    y2 = tcmid(p[0], p[1], y1, d0, d1, b1.reshape(1, D), W2)
    p = _prop(y2, src, dst, zeros_p)
    y3 = tcmid(p[0], p[1], y2, d0, d1, b2.reshape(1, D), W3)
    p = _prop(y3, src, dst, zeros_p)
    y4 = tcpre4(p[0], p[1], y3, d0, d1, b3.reshape(1, D))
    p = _prop(y4, src, dst, zeros_p)
    o = tcfin(p[0], p[1], y4, d0, d1, b4p, W4p)
    return o[:N, :2]


# speedup vs baseline: 22.6877x; 1.9734x over previous
import functools

import jax
import jax.numpy as jnp
from jax import lax
from jax.experimental import pallas as pl
from jax.experimental.pallas import tpu as pltpu
from jax.experimental.pallas import tpu_sc as plsc

N = 10000
NP = 10240
E = 320000
D = 128

NC = 2
NS = 16
CHUNK = 128
EPC = E // NC
NCHUNKS = EPC // CHUNK
ROWS_PER_TILE = NP // NS
DEGW = 128

assert EPC % CHUNK == 0 and NP % NS == 0 and ROWS_PER_TILE % 8 == 0

_MESH = dict(core_axis_name="c", subcore_axis_name="s",
             num_cores=NC, num_subcores=NS)


NCH_T = 80
NB = 2
NI = 8
_G = NCH_T // NI


@functools.partial(
    pl.kernel,
    out_type=jax.ShapeDtypeStruct((NC, NP, D), jnp.float32),
    mesh=plsc.VectorSubcoreMesh(**_MESH),
    scratch_types=[
        pltpu.VMEM_SHARED((NP, D), jnp.float32),
        pltpu.VMEM((NI, 2, CHUNK), jnp.int32),
        pltpu.VMEM((NB, CHUNK, D), jnp.float32),
        [pltpu.SemaphoreType.DMA] * NI,
        [pltpu.SemaphoreType.DMA] * NB,
    ],
)
def _prop(y_hbm, ei_hbm, zeros_hbm, out_hbm, acc, idxv, rows, isems, gsems):
    c = lax.axis_index("c")
    s = lax.axis_index("s")
    r0 = s * ROWS_PER_TILE
    nt = (NCHUNKS - s + NS - 1) // NS
    ebase = c * EPC

    def eoff(j):
        return ebase + (s + j * NS) * CHUNK

    def load_idx(slot, j):
        pltpu.async_copy(ei_hbm.at[:, pl.ds(eoff(j), CHUNK)],
                         idxv.at[slot], isems[slot])

    def wait_idx(slot):
        pltpu.make_async_copy(ei_hbm.at[:, pl.ds(0, CHUNK)],
                              idxv.at[slot], isems[slot]).wait()

    def start_gather(slot, b):
        pltpu.async_copy(y_hbm.at[idxv.at[slot, 0]], rows.at[b], gsems[b])

    def wait_gather(b):
        pltpu.make_async_copy(y_hbm.at[idxv.at[0, 0]], rows.at[b],
                              gsems[b]).wait()

    for u in range(NI):
        load_idx(u, u)

    pltpu.sync_copy(zeros_hbm, acc.at[pl.ds(r0, ROWS_PER_TILE)])
    plsc.subcore_barrier()

    for b in range(NB):
        wait_idx(b)
        start_gather(b, b)

    def body(g, carry):
        for u in range(NI):
            j = g * NI + u
            b = u % NB

            @pl.when(j < nt)
            def _(u=u, b=b, j=j):
                wait_gather(b)
                pltpu.sync_copy(rows.at[b], acc.at[idxv.at[u, 1]], add=True)

                @pl.when(j + NI < nt)
                def _(u=u, j=j):
                    load_idx(u, j + NI)

                @pl.when(j + NB < nt)
                def _(u=u, b=b):
                    un = (u + NB) % NI
                    wait_idx(un)
                    start_gather(un, b)
        return carry

    lax.fori_loop(0, _G, body, 0)
    plsc.subcore_barrier()
    pltpu.sync_copy(acc.at[pl.ds(r0, ROWS_PER_TILE)],
                    out_hbm.at[c, pl.ds(r0, ROWS_PER_TILE)])


@functools.partial(
    pl.kernel,
    out_type=jax.ShapeDtypeStruct((NC, NP, DEGW), jnp.float32),
    mesh=plsc.VectorSubcoreMesh(**_MESH),
    scratch_types=[
        pltpu.VMEM_SHARED((NP, DEGW), jnp.float32),
        pltpu.VMEM((NCH_T, CHUNK), jnp.int32),
        pltpu.VMEM((CHUNK, DEGW), jnp.float32),
        pltpu.SemaphoreType.DMA,
    ],
)
def _deg(ei_hbm, ones_hbm, zeros_hbm, out_hbm, acc, idxv, onesv, psem):
    c = lax.axis_index("c")
    s = lax.axis_index("s")
    r0 = s * ROWS_PER_TILE
    nt = (NCHUNKS - s + NS - 1) // NS
    ebase = c * EPC

    for k0 in range(0, NCH_T, 16):
        descs = []
        for k in range(k0, k0 + 16):
            off = ebase + (s + k * NS) * CHUNK
            if k >= NCH_T - 2:
                @pl.when(k < nt)
                def _(k=k, off=off):
                    pltpu.async_copy(ei_hbm.at[1, pl.ds(off, CHUNK)],
                                     idxv.at[k], psem)
            else:
                descs.append(pltpu.async_copy(
                    ei_hbm.at[1, pl.ds(off, CHUNK)], idxv.at[k], psem))
        for d in descs:
            d.wait()
        for k in range(max(k0, NCH_T - 2), k0 + 16):
            @pl.when(k < nt)
            def _(k=k):
                pltpu.make_async_copy(ei_hbm.at[1, pl.ds(0, CHUNK)],
                                      idxv.at[k], psem).wait()

    pltpu.sync_copy(ones_hbm, onesv)
    pltpu.sync_copy(zeros_hbm, acc.at[pl.ds(r0, ROWS_PER_TILE)])
    plsc.subcore_barrier()

    def body(i, carry):
        pltpu.sync_copy(onesv, acc.at[idxv.at[i]], add=True)
        return carry

    lax.fori_loop(0, nt, body, 0)
    plsc.subcore_barrier()
    pltpu.sync_copy(acc.at[pl.ds(r0, ROWS_PER_TILE)],
                    out_hbm.at[c, pl.ds(r0, ROWS_PER_TILE)])


BLK = 2048
_GRID = NP // BLK


def _dinv(d0, d1):
    return lax.rsqrt(d0[:, :1] + d1[:, :1] + 1.0)


def _tc1_body(x_ref, w_ref, d0_ref, d1_ref, y_ref):
    dinv = _dinv(d0_ref[...], d1_ref[...])
    y_ref[...] = dinv * jnp.dot(x_ref[...], w_ref[...],
                                preferred_element_type=jnp.float32)


def _tcmid_body(p0_ref, p1_ref, yp_ref, d0_ref, d1_ref, b_ref, w_ref, y_ref):
    dinv = _dinv(d0_ref[...], d1_ref[...])
    h = jnp.maximum(dinv * (p0_ref[...] + p1_ref[...] + yp_ref[...])
                    + b_ref[...], 0.0)
    y_ref[...] = dinv * jnp.dot(h, w_ref[...],
                                preferred_element_type=jnp.float32)


def _tcpre4_body(p0_ref, p1_ref, yp_ref, d0_ref, d1_ref, b_ref, y_ref):
    dinv = _dinv(d0_ref[...], d1_ref[...])
    y_ref[...] = dinv * jnp.maximum(
        dinv * (p0_ref[...] + p1_ref[...] + yp_ref[...]) + b_ref[...], 0.0)


def _tcfin_body(p0_ref, p1_ref, yp_ref, d0_ref, d1_ref, b_ref, w_ref, o_ref):
    dinv = _dinv(d0_ref[...], d1_ref[...])
    g = dinv * (p0_ref[...] + p1_ref[...] + yp_ref[...])
    o = jnp.dot(g, w_ref[...], preferred_element_type=jnp.float32) + b_ref[...]
    m = jnp.maximum(o[:, :1], o[:, 1:2])
    lse = m + jnp.log(jnp.exp(o[:, :1] - m) + jnp.exp(o[:, 1:2] - m))
    o_ref[...] = o - lse


def _row_spec(w):
    return pl.BlockSpec((BLK, w), lambda i: (i, 0))


def _full_spec(r, w):
    return pl.BlockSpec((r, w), lambda i: (0, 0))


def _tc_call(body, in_specs, out_w=D):
    return pl.pallas_call(
        body,
        grid=(_GRID,),
        in_specs=in_specs,
        out_specs=_row_spec(out_w),
        out_shape=jax.ShapeDtypeStruct((NP, out_w), jnp.float32),
    )


def kernel(x, edge_index, batch, W1, b1, W2, b2, W3, b3, W4, b4):
    xp = jnp.pad(x, ((0, NP - N), (0, 0)))
    zeros_p = jnp.zeros((ROWS_PER_TILE, D), jnp.float32)
    zeros_d = jnp.zeros((ROWS_PER_TILE, DEGW), jnp.float32)
    ones_d = jnp.ones((CHUNK, DEGW), jnp.float32)
    W4p = jnp.zeros((D, D), jnp.float32).at[:, :2].set(W4)
    b4p = jnp.zeros((1, D), jnp.float32).at[0, :2].set(b4)

    degp = _deg(edge_index, ones_d, zeros_d)
    d0, d1 = degp[0], degp[1]

    tc1 = _tc_call(_tc1_body,
                   [_row_spec(D), _full_spec(D, D),
                    _row_spec(DEGW), _row_spec(DEGW)])
    tcmid = _tc_call(_tcmid_body,
                     [_row_spec(D)] * 3 + [_row_spec(DEGW)] * 2
                     + [_full_spec(1, D), _full_spec(D, D)])
    tcpre4 = _tc_call(_tcpre4_body,
                      [_row_spec(D)] * 3 + [_row_spec(DEGW)] * 2
                      + [_full_spec(1, D)])
    tcfin = _tc_call(_tcfin_body,
                     [_row_spec(D)] * 3 + [_row_spec(DEGW)] * 2
                     + [_full_spec(1, D), _full_spec(D, D)])

    y1 = tc1(xp, W1, d0, d1)
    p = _prop(y1, edge_index, zeros_p)
    y2 = tcmid(p[0], p[1], y1, d0, d1, b1.reshape(1, D), W2)
    p = _prop(y2, edge_index, zeros_p)
    y3 = tcmid(p[0], p[1], y2, d0, d1, b2.reshape(1, D), W3)
    p = _prop(y3, edge_index, zeros_p)
    y4 = tcpre4(p[0], p[1], y3, d0, d1, b3.reshape(1, D))
    p = _prop(y4, edge_index, zeros_p)
    o = tcfin(p[0], p[1], y4, d0, d1, b4p, W4p)
    return o[:N, :2]


# speedup vs baseline: 22.7138x; 1.0012x over previous
import functools

import jax
import jax.numpy as jnp
from jax import lax
from jax.experimental import pallas as pl
from jax.experimental.pallas import tpu as pltpu
from jax.experimental.pallas import tpu_sc as plsc

N = 10000
NP = 10240
E = 320000
D = 128

NC = 2
NS = 16
CHUNK = 128
EPC = E // NC
NCHUNKS = EPC // CHUNK
ROWS_PER_TILE = NP // NS
DEGW = 128

assert EPC % CHUNK == 0 and NP % NS == 0 and ROWS_PER_TILE % 8 == 0

_MESH = dict(core_axis_name="c", subcore_axis_name="s",
             num_cores=NC, num_subcores=NS)


NCH_T = 80
NB = 2
NI = 8
_G = NCH_T // NI


@functools.partial(
    pl.kernel,
    out_type=jax.ShapeDtypeStruct((NC, NP, D), jnp.float32),
    mesh=plsc.VectorSubcoreMesh(**_MESH),
    scratch_types=[
        pltpu.VMEM_SHARED((NP, D), jnp.float32),
        pltpu.VMEM((NI, 2, CHUNK), jnp.int32),
        pltpu.VMEM((NB, CHUNK, D), jnp.float32),
        [pltpu.SemaphoreType.DMA] * NI,
        [pltpu.SemaphoreType.DMA] * NB,
    ],
)
def _prop(y_hbm, ei_hbm, zeros_hbm, out_hbm, acc, idxv, rows, isems, gsems):
    c = lax.axis_index("c")
    s = lax.axis_index("s")
    r0 = s * ROWS_PER_TILE
    nt = (NCHUNKS - s + NS - 1) // NS
    ebase = c * EPC

    def eoff(j):
        return ebase + (s + j * NS) * CHUNK

    def load_idx(slot, j):
        pltpu.async_copy(ei_hbm.at[:, pl.ds(eoff(j), CHUNK)],
                         idxv.at[slot], isems[slot])

    def wait_idx(slot):
        pltpu.make_async_copy(ei_hbm.at[:, pl.ds(0, CHUNK)],
                              idxv.at[slot], isems[slot]).wait()

    def start_gather(slot, b):
        pltpu.async_copy(y_hbm.at[idxv.at[slot, 0]], rows.at[b], gsems[b])

    def wait_gather(b):
        pltpu.make_async_copy(y_hbm.at[idxv.at[0, 0]], rows.at[b],
                              gsems[b]).wait()

    for u in range(NI):
        load_idx(u, u)

    for b in range(NB):
        wait_idx(b)
        start_gather(b, b)

    pltpu.sync_copy(zeros_hbm, acc.at[pl.ds(r0, ROWS_PER_TILE)])
    plsc.subcore_barrier()

    def body(g, carry):
        for u in range(NI):
            j = g * NI + u
            b = u % NB

            @pl.when(j < nt)
            def _(u=u, b=b, j=j):
                wait_gather(b)
                pltpu.sync_copy(rows.at[b], acc.at[idxv.at[u, 1]], add=True)

                @pl.when(j + NI < nt)
                def _(u=u, j=j):
                    load_idx(u, j + NI)

                @pl.when(j + NB < nt)
                def _(u=u, b=b):
                    un = (u + NB) % NI
                    wait_idx(un)
                    start_gather(un, b)
        return carry

    lax.fori_loop(0, _G, body, 0)
    plsc.subcore_barrier()
    pltpu.sync_copy(acc.at[pl.ds(r0, ROWS_PER_TILE)],
                    out_hbm.at[c, pl.ds(r0, ROWS_PER_TILE)])


@functools.partial(
    pl.kernel,
    out_type=jax.ShapeDtypeStruct((NC, NP, DEGW), jnp.float32),
    mesh=plsc.VectorSubcoreMesh(**_MESH),
    scratch_types=[
        pltpu.VMEM_SHARED((NP, DEGW), jnp.float32),
        pltpu.VMEM((NCH_T, CHUNK), jnp.int32),
        pltpu.VMEM((CHUNK, DEGW), jnp.float32),
        pltpu.SemaphoreType.DMA,
        pltpu.SemaphoreType.DMA,
    ],
)
def _deg(ei_hbm, ones_hbm, zeros_hbm, out_hbm, acc, idxv, onesv, psem, ssem):
    c = lax.axis_index("c")
    s = lax.axis_index("s")
    r0 = s * ROWS_PER_TILE
    nt = (NCHUNKS - s + NS - 1) // NS
    ebase = c * EPC

    for k0 in range(0, NCH_T, 16):
        descs = []
        for k in range(k0, k0 + 16):
            off = ebase + (s + k * NS) * CHUNK
            if k >= NCH_T - 2:
                @pl.when(k < nt)
                def _(k=k, off=off):
                    pltpu.async_copy(ei_hbm.at[1, pl.ds(off, CHUNK)],
                                     idxv.at[k], psem)
            else:
                descs.append(pltpu.async_copy(
                    ei_hbm.at[1, pl.ds(off, CHUNK)], idxv.at[k], psem))
        for d in descs:
            d.wait()
        for k in range(max(k0, NCH_T - 2), k0 + 16):
            @pl.when(k < nt)
            def _(k=k):
                pltpu.make_async_copy(ei_hbm.at[1, pl.ds(0, CHUNK)],
                                      idxv.at[k], psem).wait()

    pltpu.sync_copy(ones_hbm, onesv)
    pltpu.sync_copy(zeros_hbm, acc.at[pl.ds(r0, ROWS_PER_TILE)])
    plsc.subcore_barrier()

    GRP = 16
    for g0 in range(0, NCH_T, GRP):
        for j in range(g0, g0 + GRP):
            @pl.when(j < nt)
            def _(j=j):
                pltpu.async_copy(onesv, acc.at[idxv.at[j]], ssem, add=True)
        if g0 > 0:
            for j in range(g0 - GRP, g0):
                @pl.when(j < nt)
                def _(j=j):
                    pltpu.make_async_copy(ones_hbm, onesv, ssem).wait()
    for j in range(NCH_T - GRP, NCH_T):
        @pl.when(j < nt)
        def _(j=j):
            pltpu.make_async_copy(ones_hbm, onesv, ssem).wait()
    plsc.subcore_barrier()
    pltpu.sync_copy(acc.at[pl.ds(r0, ROWS_PER_TILE)],
                    out_hbm.at[c, pl.ds(r0, ROWS_PER_TILE)])


BLK = 2048
_GRID = NP // BLK


def _dinv(d0, d1):
    return lax.rsqrt(d0[:, :1] + d1[:, :1] + 1.0)


def _tc1_body(x_ref, w_ref, d0_ref, d1_ref, y_ref):
    dinv = _dinv(d0_ref[...], d1_ref[...])
    y_ref[...] = dinv * jnp.dot(x_ref[...], w_ref[...],
                                preferred_element_type=jnp.float32)


def _tcmid_body(p0_ref, p1_ref, yp_ref, d0_ref, d1_ref, b_ref, w_ref, y_ref):
    dinv = _dinv(d0_ref[...], d1_ref[...])
    h = jnp.maximum(dinv * (p0_ref[...] + p1_ref[...] + yp_ref[...])
                    + b_ref[...], 0.0)
    y_ref[...] = dinv * jnp.dot(h, w_ref[...],
                                preferred_element_type=jnp.float32)


def _tcpre4_body(p0_ref, p1_ref, yp_ref, d0_ref, d1_ref, b_ref, y_ref):
    dinv = _dinv(d0_ref[...], d1_ref[...])
    y_ref[...] = dinv * jnp.maximum(
        dinv * (p0_ref[...] + p1_ref[...] + yp_ref[...]) + b_ref[...], 0.0)


def _tcfin_body(p0_ref, p1_ref, yp_ref, d0_ref, d1_ref, b_ref, w_ref, o_ref):
    dinv = _dinv(d0_ref[...], d1_ref[...])
    g = dinv * (p0_ref[...] + p1_ref[...] + yp_ref[...])
    o = jnp.dot(g, w_ref[...], preferred_element_type=jnp.float32) + b_ref[...]
    m = jnp.maximum(o[:, :1], o[:, 1:2])
    lse = m + jnp.log(jnp.exp(o[:, :1] - m) + jnp.exp(o[:, 1:2] - m))
    o_ref[...] = o - lse


def _row_spec(w):
    return pl.BlockSpec((BLK, w), lambda i: (i, 0))


def _full_spec(r, w):
    return pl.BlockSpec((r, w), lambda i: (0, 0))


def _tc_call(body, in_specs, out_w=D):
    return pl.pallas_call(
        body,
        grid=(_GRID,),
        in_specs=in_specs,
        out_specs=_row_spec(out_w),
        out_shape=jax.ShapeDtypeStruct((NP, out_w), jnp.float32),
    )


def kernel(x, edge_index, batch, W1, b1, W2, b2, W3, b3, W4, b4):
    xp = jnp.pad(x, ((0, NP - N), (0, 0)))
    zeros_p = jnp.zeros((ROWS_PER_TILE, D), jnp.float32)
    zeros_d = jnp.zeros((ROWS_PER_TILE, DEGW), jnp.float32)
    ones_d = jnp.ones((CHUNK, DEGW), jnp.float32)
    W4p = jnp.zeros((D, D), jnp.float32).at[:, :2].set(W4)
    b4p = jnp.zeros((1, D), jnp.float32).at[0, :2].set(b4)

    degp = _deg(edge_index, ones_d, zeros_d)
    d0, d1 = degp[0], degp[1]

    tc1 = _tc_call(_tc1_body,
                   [_row_spec(D), _full_spec(D, D),
                    _row_spec(DEGW), _row_spec(DEGW)])
    tcmid = _tc_call(_tcmid_body,
                     [_row_spec(D)] * 3 + [_row_spec(DEGW)] * 2
                     + [_full_spec(1, D), _full_spec(D, D)])
    tcpre4 = _tc_call(_tcpre4_body,
                      [_row_spec(D)] * 3 + [_row_spec(DEGW)] * 2
                      + [_full_spec(1, D)])
    tcfin = _tc_call(_tcfin_body,
                     [_row_spec(D)] * 3 + [_row_spec(DEGW)] * 2
                     + [_full_spec(1, D), _full_spec(D, D)])

    y1 = tc1(xp, W1, d0, d1)
    p = _prop(y1, edge_index, zeros_p)
    y2 = tcmid(p[0], p[1], y1, d0, d1, b1.reshape(1, D), W2)
    p = _prop(y2, edge_index, zeros_p)
    y3 = tcmid(p[0], p[1], y2, d0, d1, b2.reshape(1, D), W3)
    p = _prop(y3, edge_index, zeros_p)
    y4 = tcpre4(p[0], p[1], y3, d0, d1, b3.reshape(1, D))
    p = _prop(y4, edge_index, zeros_p)
    o = tcfin(p[0], p[1], y4, d0, d1, b4p, W4p)
    return o[:N, :2]


# speedup vs baseline: 24.0899x; 1.0606x over previous
import functools

import jax
import jax.numpy as jnp
from jax import lax
from jax.experimental import pallas as pl
from jax.experimental.pallas import tpu as pltpu
from jax.experimental.pallas import tpu_sc as plsc

N = 10000
NP = 10240
E = 320000
D = 128

NC = 2
NS = 16
CHUNK = 128
EPC = E // NC
NCHUNKS = EPC // CHUNK
ROWS_PER_TILE = NP // NS
DEGW = 128

assert EPC % CHUNK == 0 and NP % NS == 0 and ROWS_PER_TILE % 8 == 0

_MESH = dict(core_axis_name="c", subcore_axis_name="s",
             num_cores=NC, num_subcores=NS)


NCH_T = 80
NB = 2
NI = 8
_G = NCH_T // NI


@functools.partial(
    pl.kernel,
    out_type=jax.ShapeDtypeStruct((NC, NP, D), jnp.float32),
    mesh=plsc.VectorSubcoreMesh(**_MESH),
    scratch_types=[
        pltpu.VMEM_SHARED((NP, D), jnp.float32),
        pltpu.VMEM((NI, 2, CHUNK), jnp.int32),
        pltpu.VMEM((NB, CHUNK, D), jnp.float32),
        [pltpu.SemaphoreType.DMA] * NI,
        [pltpu.SemaphoreType.DMA] * NB,
    ],
)
def _prop(y_hbm, ei_hbm, zeros_hbm, out_hbm, acc, idxv, rows, isems, gsems):
    c = lax.axis_index("c")
    s = lax.axis_index("s")
    r0 = s * ROWS_PER_TILE
    nt = (NCHUNKS - s + NS - 1) // NS
    ebase = c * EPC

    def eoff(j):
        return ebase + (s + j * NS) * CHUNK

    def load_idx(slot, j):
        pltpu.async_copy(ei_hbm.at[:, pl.ds(eoff(j), CHUNK)],
                         idxv.at[slot], isems[slot])

    def wait_idx(slot):
        pltpu.make_async_copy(ei_hbm.at[:, pl.ds(0, CHUNK)],
                              idxv.at[slot], isems[slot]).wait()

    def start_gather(slot, b):
        pltpu.async_copy(y_hbm.at[idxv.at[slot, 0]], rows.at[b], gsems[b])

    def wait_gather(b):
        pltpu.make_async_copy(y_hbm.at[idxv.at[0, 0]], rows.at[b],
                              gsems[b]).wait()

    for u in range(NI):
        load_idx(u, u)

    for b in range(NB):
        wait_idx(b)
        start_gather(b, b)

    pltpu.sync_copy(zeros_hbm, acc.at[pl.ds(r0, ROWS_PER_TILE)])
    plsc.subcore_barrier()

    def body(g, carry):
        for u in range(NI):
            j = g * NI + u
            b = u % NB

            @pl.when(j < nt)
            def _(u=u, b=b, j=j):
                wait_gather(b)
                pltpu.sync_copy(rows.at[b], acc.at[idxv.at[u, 1]], add=True)

                @pl.when(j + NI < nt)
                def _(u=u, j=j):
                    load_idx(u, j + NI)

                @pl.when(j + NB < nt)
                def _(u=u, b=b):
                    un = (u + NB) % NI
                    wait_idx(un)
                    start_gather(un, b)
        return carry

    lax.fori_loop(0, _G, body, 0)
    plsc.subcore_barrier()
    pltpu.sync_copy(acc.at[pl.ds(r0, ROWS_PER_TILE)],
                    out_hbm.at[c, pl.ds(r0, ROWS_PER_TILE)])


@functools.partial(
    pl.kernel,
    out_type=jax.ShapeDtypeStruct((NC, NP, DEGW), jnp.float32),
    mesh=plsc.VectorSubcoreMesh(**_MESH),
    scratch_types=[
        pltpu.VMEM_SHARED((NP, DEGW), jnp.float32),
        pltpu.VMEM((NCH_T, CHUNK), jnp.int32),
        pltpu.VMEM((CHUNK, DEGW), jnp.float32),
        pltpu.SemaphoreType.DMA,
        pltpu.SemaphoreType.DMA,
    ],
)
def _deg(ei_hbm, ones_hbm, zeros_hbm, out_hbm, acc, idxv, onesv, psem, ssem):
    c = lax.axis_index("c")
    s = lax.axis_index("s")
    r0 = s * ROWS_PER_TILE
    nt = (NCHUNKS - s + NS - 1) // NS
    ebase = c * EPC

    for k0 in range(0, NCH_T, 16):
        descs = []
        for k in range(k0, k0 + 16):
            off = ebase + (s + k * NS) * CHUNK
            if k >= NCH_T - 2:
                @pl.when(k < nt)
                def _(k=k, off=off):
                    pltpu.async_copy(ei_hbm.at[1, pl.ds(off, CHUNK)],
                                     idxv.at[k], psem)
            else:
                descs.append(pltpu.async_copy(
                    ei_hbm.at[1, pl.ds(off, CHUNK)], idxv.at[k], psem))
        for d in descs:
            d.wait()
        for k in range(max(k0, NCH_T - 2), k0 + 16):
            @pl.when(k < nt)
            def _(k=k):
                pltpu.make_async_copy(ei_hbm.at[1, pl.ds(0, CHUNK)],
                                      idxv.at[k], psem).wait()

    pltpu.sync_copy(ones_hbm, onesv)
    pltpu.sync_copy(zeros_hbm, acc.at[pl.ds(r0, ROWS_PER_TILE)])
    plsc.subcore_barrier()

    GRP = 16
    for g0 in range(0, NCH_T, GRP):
        for j in range(g0, g0 + GRP):
            @pl.when(j < nt)
            def _(j=j):
                pltpu.async_copy(onesv, acc.at[idxv.at[j]], ssem, add=True)
        if g0 > 0:
            for j in range(g0 - GRP, g0):
                @pl.when(j < nt)
                def _(j=j):
                    pltpu.make_async_copy(ones_hbm, onesv, ssem).wait()
    for j in range(NCH_T - GRP, NCH_T):
        @pl.when(j < nt)
        def _(j=j):
            pltpu.make_async_copy(ones_hbm, onesv, ssem).wait()
    plsc.subcore_barrier()
    pltpu.sync_copy(acc.at[pl.ds(r0, ROWS_PER_TILE)],
                    out_hbm.at[c, pl.ds(r0, ROWS_PER_TILE)])


BLK = 2048
_GRID = NP // BLK


def _dinv(d_ref):
    return lax.rsqrt(d_ref[0, :, :1] + d_ref[1, :, :1] + 1.0)


def _tc1_body(x_ref, w_ref, d_ref, y_ref):
    dinv = _dinv(d_ref)
    y_ref[...] = dinv * jnp.dot(x_ref[...], w_ref[...],
                                preferred_element_type=jnp.float32)


def _tcmid_body(p_ref, yp_ref, d_ref, b_ref, w_ref, y_ref):
    dinv = _dinv(d_ref)
    h = jnp.maximum(dinv * (p_ref[0] + p_ref[1] + yp_ref[...])
                    + b_ref[...], 0.0)
    y_ref[...] = dinv * jnp.dot(h, w_ref[...],
                                preferred_element_type=jnp.float32)


def _tcpre4_body(p_ref, yp_ref, d_ref, b_ref, y_ref):
    dinv = _dinv(d_ref)
    y_ref[...] = dinv * jnp.maximum(
        dinv * (p_ref[0] + p_ref[1] + yp_ref[...]) + b_ref[...], 0.0)


def _tcfin_body(p_ref, yp_ref, d_ref, b_ref, w_ref, o_ref):
    dinv = _dinv(d_ref)
    g = dinv * (p_ref[0] + p_ref[1] + yp_ref[...])
    o = jnp.dot(g, w_ref[...], preferred_element_type=jnp.float32) + b_ref[...]
    m = jnp.maximum(o[:, :1], o[:, 1:2])
    lse = m + jnp.log(jnp.exp(o[:, :1] - m) + jnp.exp(o[:, 1:2] - m))
    o_ref[...] = o - lse


def _row_spec(w):
    return pl.BlockSpec((BLK, w), lambda i: (i, 0))


def _pair_spec():
    return pl.BlockSpec((NC, BLK, D), lambda i: (0, i, 0))


def _full_spec(r, w):
    return pl.BlockSpec((r, w), lambda i: (0, 0))


def _tc_call(body, in_specs, out_w=D):
    return pl.pallas_call(
        body,
        grid=(_GRID,),
        in_specs=in_specs,
        out_specs=_row_spec(out_w),
        out_shape=jax.ShapeDtypeStruct((NP, out_w), jnp.float32),
    )


def kernel(x, edge_index, batch, W1, b1, W2, b2, W3, b3, W4, b4):
    xp = jnp.pad(x, ((0, NP - N), (0, 0)))
    zeros_p = jnp.zeros((ROWS_PER_TILE, D), jnp.float32)
    zeros_d = jnp.zeros((ROWS_PER_TILE, DEGW), jnp.float32)
    ones_d = jnp.ones((CHUNK, DEGW), jnp.float32)
    W4p = jnp.zeros((D, D), jnp.float32).at[:, :2].set(W4)
    b4p = jnp.zeros((1, D), jnp.float32).at[0, :2].set(b4)

    degp = _deg(edge_index, ones_d, zeros_d)

    tc1 = _tc_call(_tc1_body,
                   [_row_spec(D), _full_spec(D, D), _pair_spec()])
    tcmid = _tc_call(_tcmid_body,
                     [_pair_spec(), _row_spec(D), _pair_spec(),
                      _full_spec(1, D), _full_spec(D, D)])
    tcpre4 = _tc_call(_tcpre4_body,
                      [_pair_spec(), _row_spec(D), _pair_spec(),
                       _full_spec(1, D)])
    tcfin = _tc_call(_tcfin_body,
                     [_pair_spec(), _row_spec(D), _pair_spec(),
                      _full_spec(1, D), _full_spec(D, D)])

    y1 = tc1(xp, W1, degp)
    p = _prop(y1, edge_index, zeros_p)
    y2 = tcmid(p, y1, degp, b1.reshape(1, D), W2)
    p = _prop(y2, edge_index, zeros_p)
    y3 = tcmid(p, y2, degp, b2.reshape(1, D), W3)
    p = _prop(y3, edge_index, zeros_p)
    y4 = tcpre4(p, y3, degp, b3.reshape(1, D))
    p = _prop(y4, edge_index, zeros_p)
    o = tcfin(p, y4, degp, b4p, W4p)
    return o[:N, :2]


# speedup vs baseline: 24.2203x; 1.0054x over previous
import functools

import jax
import jax.numpy as jnp
from jax import lax
from jax.experimental import pallas as pl
from jax.experimental.pallas import tpu as pltpu
from jax.experimental.pallas import tpu_sc as plsc

N = 10000
NP = 10240
E = 320000
D = 128

NC = 2
NS = 16
CHUNK = 128
EPC = E // NC
NCHUNKS = EPC // CHUNK
ROWS_PER_TILE = NP // NS
DEGW = 128

assert EPC % CHUNK == 0 and NP % NS == 0 and ROWS_PER_TILE % 8 == 0

_MESH = dict(core_axis_name="c", subcore_axis_name="s",
             num_cores=NC, num_subcores=NS)


NCH_T = 80
NB = 2
NI = 8
_G = NCH_T // NI


@functools.partial(
    pl.kernel,
    out_type=jax.ShapeDtypeStruct((NC, NP, D), jnp.float32),
    mesh=plsc.VectorSubcoreMesh(**_MESH),
    scratch_types=[
        pltpu.VMEM_SHARED((NP, D), jnp.float32),
        pltpu.VMEM((NI, 2, CHUNK), jnp.int32),
        pltpu.VMEM((NB, CHUNK, D), jnp.float32),
        [pltpu.SemaphoreType.DMA] * NI,
        [pltpu.SemaphoreType.DMA] * NB,
    ],
)
def _prop(y_hbm, ei_hbm, zeros_hbm, out_hbm, acc, idxv, rows, isems, gsems):
    c = lax.axis_index("c")
    s = lax.axis_index("s")
    r0 = s * ROWS_PER_TILE
    nt = (NCHUNKS - s + NS - 1) // NS
    ebase = c * EPC

    def eoff(j):
        return ebase + (s + j * NS) * CHUNK

    def load_idx(slot, j):
        pltpu.async_copy(ei_hbm.at[:, pl.ds(eoff(j), CHUNK)],
                         idxv.at[slot], isems[slot])

    def wait_idx(slot):
        pltpu.make_async_copy(ei_hbm.at[:, pl.ds(0, CHUNK)],
                              idxv.at[slot], isems[slot]).wait()

    def start_gather(slot, b):
        pltpu.async_copy(y_hbm.at[idxv.at[slot, 0]], rows.at[b], gsems[b])

    def wait_gather(b):
        pltpu.make_async_copy(y_hbm.at[idxv.at[0, 0]], rows.at[b],
                              gsems[b]).wait()

    for u in range(NI):
        load_idx(u, u)

    for b in range(NB):
        wait_idx(b)
        start_gather(b, b)

    pltpu.sync_copy(zeros_hbm, acc.at[pl.ds(r0, ROWS_PER_TILE)])
    plsc.subcore_barrier()

    def body(g, carry):
        for u in range(NI):
            j = g * NI + u
            b = u % NB

            @pl.when(j < nt)
            def _(u=u, b=b, j=j):
                wait_gather(b)
                pltpu.sync_copy(rows.at[b], acc.at[idxv.at[u, 1]], add=True)

                @pl.when(j + NI < nt)
                def _(u=u, j=j):
                    load_idx(u, j + NI)

                @pl.when(j + NB < nt)
                def _(u=u, b=b):
                    un = (u + NB) % NI
                    wait_idx(un)
                    start_gather(un, b)
        return carry

    lax.fori_loop(0, _G, body, 0)
    plsc.subcore_barrier()
    pltpu.sync_copy(acc.at[pl.ds(r0, ROWS_PER_TILE)],
                    out_hbm.at[c, pl.ds(r0, ROWS_PER_TILE)])


@functools.partial(
    pl.kernel,
    out_type=jax.ShapeDtypeStruct((NC, NP, DEGW), jnp.float32),
    mesh=plsc.VectorSubcoreMesh(**_MESH),
    scratch_types=[
        pltpu.VMEM_SHARED((NP, DEGW), jnp.float32),
        pltpu.VMEM((NCH_T, CHUNK), jnp.int32),
        pltpu.VMEM((CHUNK, DEGW), jnp.float32),
        pltpu.SemaphoreType.DMA,
        pltpu.SemaphoreType.DMA,
    ],
)
def _deg(ei_hbm, ones_hbm, zeros_hbm, out_hbm, acc, idxv, onesv, psem, ssem):
    c = lax.axis_index("c")
    s = lax.axis_index("s")
    r0 = s * ROWS_PER_TILE
    nt = (NCHUNKS - s + NS - 1) // NS
    ebase = c * EPC

    for k0 in range(0, NCH_T, 40):
        descs = []
        for k in range(k0, k0 + 40):
            off = ebase + (s + k * NS) * CHUNK
            if k >= NCH_T - 2:
                @pl.when(k < nt)
                def _(k=k, off=off):
                    pltpu.async_copy(ei_hbm.at[1, pl.ds(off, CHUNK)],
                                     idxv.at[k], psem)
            else:
                descs.append(pltpu.async_copy(
                    ei_hbm.at[1, pl.ds(off, CHUNK)], idxv.at[k], psem))
        for d in descs:
            d.wait()
        for k in range(max(k0, NCH_T - 2), k0 + 40):
            @pl.when(k < nt)
            def _(k=k):
                pltpu.make_async_copy(ei_hbm.at[1, pl.ds(0, CHUNK)],
                                      idxv.at[k], psem).wait()

    pltpu.sync_copy(ones_hbm, onesv)
    pltpu.sync_copy(zeros_hbm, acc.at[pl.ds(r0, ROWS_PER_TILE)])
    plsc.subcore_barrier()

    GRP = 16
    for g0 in range(0, NCH_T, GRP):
        for j in range(g0, g0 + GRP):
            @pl.when(j < nt)
            def _(j=j):
                pltpu.async_copy(onesv, acc.at[idxv.at[j]], ssem, add=True)
        if g0 > 0:
            for j in range(g0 - GRP, g0):
                @pl.when(j < nt)
                def _(j=j):
                    pltpu.make_async_copy(ones_hbm, onesv, ssem).wait()
    for j in range(NCH_T - GRP, NCH_T):
        @pl.when(j < nt)
        def _(j=j):
            pltpu.make_async_copy(ones_hbm, onesv, ssem).wait()
    plsc.subcore_barrier()
    pltpu.sync_copy(acc.at[pl.ds(r0, ROWS_PER_TILE)],
                    out_hbm.at[c, pl.ds(r0, ROWS_PER_TILE)])


BLK = 2048
_GRID = NP // BLK


def _tc1_body(x_ref, w_ref, d_ref, y_ref, dinv_ref):
    dinv = lax.rsqrt(d_ref[0, :, :1] + d_ref[1, :, :1] + 1.0)
    dinv_ref[...] = jnp.broadcast_to(dinv, dinv_ref.shape)
    y_ref[...] = dinv * jnp.dot(x_ref[...], w_ref[...],
                                preferred_element_type=jnp.float32)


def _tcmid_body(p_ref, yp_ref, d_ref, b_ref, w_ref, y_ref):
    dinv = d_ref[:, :1]
    h = jnp.maximum(dinv * (p_ref[0] + p_ref[1] + yp_ref[...])
                    + b_ref[...], 0.0)
    y_ref[...] = dinv * jnp.dot(h, w_ref[...],
                                preferred_element_type=jnp.float32)


def _tcpre4_body(p_ref, yp_ref, d_ref, b_ref, y_ref):
    dinv = d_ref[:, :1]
    y_ref[...] = dinv * jnp.maximum(
        dinv * (p_ref[0] + p_ref[1] + yp_ref[...]) + b_ref[...], 0.0)


def _tcfin_body(p_ref, yp_ref, d_ref, b_ref, w_ref, o_ref):
    dinv = d_ref[:, :1]
    g = dinv * (p_ref[0] + p_ref[1] + yp_ref[...])
    o = jnp.dot(g, w_ref[...], preferred_element_type=jnp.float32) + b_ref[...]
    m = jnp.maximum(o[:, :1], o[:, 1:2])
    lse = m + jnp.log(jnp.exp(o[:, :1] - m) + jnp.exp(o[:, 1:2] - m))
    o_ref[...] = o - lse


def _row_spec(w):
    return pl.BlockSpec((BLK, w), lambda i: (i, 0))


def _pair_spec():
    return pl.BlockSpec((NC, BLK, D), lambda i: (0, i, 0))


def _full_spec(r, w):
    return pl.BlockSpec((r, w), lambda i: (0, 0))


def _tc_call(body, in_specs, n_out=1):
    return pl.pallas_call(
        body,
        grid=(_GRID,),
        in_specs=in_specs,
        out_specs=[_row_spec(D)] * n_out,
        out_shape=[jax.ShapeDtypeStruct((NP, D), jnp.float32)] * n_out,
    )


def kernel(x, edge_index, batch, W1, b1, W2, b2, W3, b3, W4, b4):
    xp = jnp.pad(x, ((0, NP - N), (0, 0)))
    zeros_p = jnp.zeros((ROWS_PER_TILE, D), jnp.float32)
    zeros_d = jnp.zeros((ROWS_PER_TILE, DEGW), jnp.float32)
    ones_d = jnp.ones((CHUNK, DEGW), jnp.float32)
    W4p = jnp.zeros((D, D), jnp.float32).at[:, :2].set(W4)
    b4p = jnp.zeros((1, D), jnp.float32).at[0, :2].set(b4)

    degp = _deg(edge_index, ones_d, zeros_d)

    tc1 = _tc_call(_tc1_body,
                   [_row_spec(D), _full_spec(D, D), _pair_spec()], n_out=2)
    tcmid = _tc_call(_tcmid_body,
                     [_pair_spec(), _row_spec(D), _row_spec(D),
                      _full_spec(1, D), _full_spec(D, D)])
    tcpre4 = _tc_call(_tcpre4_body,
                      [_pair_spec(), _row_spec(D), _row_spec(D),
                       _full_spec(1, D)])
    tcfin = _tc_call(_tcfin_body,
                     [_pair_spec(), _row_spec(D), _row_spec(D),
                      _full_spec(1, D), _full_spec(D, D)])

    y1, dinv = tc1(xp, W1, degp)
    p = _prop(y1, edge_index, zeros_p)
    y2, = tcmid(p, y1, dinv, b1.reshape(1, D), W2)
    p = _prop(y2, edge_index, zeros_p)
    y3, = tcmid(p, y2, dinv, b2.reshape(1, D), W3)
    p = _prop(y3, edge_index, zeros_p)
    y4, = tcpre4(p, y3, dinv, b3.reshape(1, D))
    p = _prop(y4, edge_index, zeros_p)
    o, = tcfin(p, y4, dinv, b4p, W4p)
    return o[:N, :2]
